# baseline (speedup 1.0000x reference)
import functools

import jax
import jax.numpy as jnp
from jax import lax
from jax.experimental import pallas as pl
from jax.experimental.pallas import tpu as pltpu
from jax.experimental.pallas import tpu_sc as plsc

VOCAB = 50257
HIDDEN = 768
MAX_POS = 8192
BATCH = 4
SEQ = 2048

NUM_CORES = 2
NUM_SUBCORES = 16
NUM_WORKERS = NUM_CORES * NUM_SUBCORES
TOTAL = BATCH * SEQ
PER_WORKER = TOTAL // NUM_WORKERS
CHUNK = 16
NCHUNKS = PER_WORKER // CHUNK
LANES = 16
VECS_PER_ROW = HIDDEN // LANES
NTOK = 4
NPOS = 4
DEPTH = 3


def _emb_body(ids_hbm, tok_hbm, pos_hbm, out_hbm,
              idx_v, tok_bufs, pos_bufs, gsems, psems, osems):
    wid = lax.axis_index("s") * NUM_CORES + lax.axis_index("c")
    base = wid * PER_WORKER
    pos_base = base % SEQ

    pltpu.sync_copy(ids_hbm.at[pl.ds(base, PER_WORKER)], idx_v)

    gh = [None] * NCHUNKS
    ph = [None] * NCHUNKS
    oh = [None] * NCHUNKS

    def start_gather(c):
        tb = c % NTOK
        gh[c] = pltpu.async_copy(
            tok_hbm.at[idx_v.at[pl.ds(c * CHUNK, CHUNK)]],
            tok_bufs.at[tb], gsems.at[tb])

    def start_pos(c):
        pb = c % NPOS
        ph[c] = pltpu.async_copy(
            pos_hbm.at[pl.ds(pos_base + c * CHUNK, CHUNK)],
            pos_bufs.at[pb], psems.at[pb])

    for c in range(DEPTH):
        start_gather(c)
        start_pos(c)

    for c in range(NCHUNKS):
        tb = c % NTOK
        pb = c % NPOS
        gh[c].wait()
        ph[c].wait()

        nc = c + DEPTH
        if nc < NCHUNKS:
            start_gather(nc)
            if nc >= NPOS:
                oh[nc - NPOS].wait()
            start_pos(nc)

        @plsc.parallel_loop(0, CHUNK, unroll=2)
        def add_row(r):
            for j in range(VECS_PER_ROW):
                sl = pl.ds(j * LANES, LANES)
                plsc.addupdate(pos_bufs.at[pb, r, sl], tok_bufs[tb, r, sl])

        oh[c] = pltpu.async_copy(
            pos_bufs.at[pb], out_hbm.at[pl.ds(base + c * CHUNK, CHUNK)],
            osems.at[pb])

    for c in range(NCHUNKS - NPOS, NCHUNKS):
        oh[c].wait()


@jax.jit
def _emb(ids_flat, token_table, pos_table):
    mesh = plsc.VectorSubcoreMesh(core_axis_name="c", subcore_axis_name="s")
    k = functools.partial(
        pl.kernel,
        out_type=jax.ShapeDtypeStruct((TOTAL, HIDDEN), jnp.float32),
        mesh=mesh,
        scratch_types=[
            pltpu.VMEM((PER_WORKER,), jnp.int32),
            pltpu.VMEM((NTOK, CHUNK, HIDDEN), jnp.float32),
            pltpu.VMEM((NPOS, CHUNK, HIDDEN), jnp.float32),
            pltpu.SemaphoreType.DMA((NTOK,)),
            pltpu.SemaphoreType.DMA((NPOS,)),
            pltpu.SemaphoreType.DMA((NPOS,)),
        ],
    )(_emb_body)
    return k(ids_flat, token_table, pos_table)


def kernel(input_ids, token_table, pos_table):
    ids_flat = input_ids.reshape(-1).astype(jnp.int32)
    out = _emb(ids_flat, token_table, pos_table)
    return out.reshape(BATCH, SEQ, HIDDEN)


# speedup vs baseline: 1.1095x; 1.1095x over previous
import functools

import jax
import jax.numpy as jnp
from jax import lax
from jax.experimental import pallas as pl
from jax.experimental.pallas import tpu as pltpu
from jax.experimental.pallas import tpu_sc as plsc

VOCAB = 50257
HIDDEN = 768
MAX_POS = 8192
BATCH = 4
SEQ = 2048

NUM_CORES = 2
NUM_SUBCORES = 16
NUM_WORKERS = NUM_CORES * NUM_SUBCORES
POS_PER_WORKER = SEQ // NUM_WORKERS
PER_WORKER = BATCH * POS_PER_WORKER
TOTAL = BATCH * SEQ
CHUNK = 16
CHUNKS_PER_BATCH = POS_PER_WORKER // CHUNK
NCHUNKS = BATCH * CHUNKS_PER_BATCH
LANES = 16
VECS_PER_ROW = HIDDEN // LANES
NBUF = 6
DEPTH = 3


def _emb_body(ids_hbm, tok_hbm, pos_hbm, out_hbm,
              idx_v, pos_local, tok_bufs, gsems, osems):
    wid = lax.axis_index("s") * NUM_CORES + lax.axis_index("c")
    p0 = wid * POS_PER_WORKER

    pltpu.sync_copy(pos_hbm.at[pl.ds(p0, POS_PER_WORKER)], pos_local)
    for bi in range(BATCH):
        pltpu.sync_copy(ids_hbm.at[pl.ds(bi * SEQ + p0, POS_PER_WORKER)],
                        idx_v.at[bi])

    gh = [None] * NCHUNKS
    oh = [None] * NCHUNKS

    def start_gather(c):
        b = c % NBUF
        bi, h = divmod(c, CHUNKS_PER_BATCH)
        gh[c] = pltpu.async_copy(
            tok_hbm.at[idx_v.at[bi, pl.ds(h * CHUNK, CHUNK)]],
            tok_bufs.at[b], gsems.at[b])

    for c in range(DEPTH):
        start_gather(c)

    for c in range(NCHUNKS):
        b = c % NBUF
        bi, h = divmod(c, CHUNKS_PER_BATCH)
        gh[c].wait()

        nc = c + DEPTH
        if nc < NCHUNKS:
            if nc >= NBUF:
                oh[nc - NBUF].wait()
            start_gather(nc)

        @plsc.parallel_loop(0, CHUNK)
        def add_row(r):
            for j in range(VECS_PER_ROW):
                sl = pl.ds(j * LANES, LANES)
                plsc.addupdate(tok_bufs.at[b, r, sl],
                               pos_local[h * CHUNK + r, sl])

        oh[c] = pltpu.async_copy(
            tok_bufs.at[b],
            out_hbm.at[pl.ds(bi * SEQ + p0 + h * CHUNK, CHUNK)],
            osems.at[b])

    for c in range(NCHUNKS - NBUF, NCHUNKS):
        oh[c].wait()


@jax.jit
def _emb(ids_flat, token_table, pos_table):
    mesh = plsc.VectorSubcoreMesh(core_axis_name="c", subcore_axis_name="s")
    k = functools.partial(
        pl.kernel,
        out_type=jax.ShapeDtypeStruct((TOTAL, HIDDEN), jnp.float32),
        mesh=mesh,
        scratch_types=[
            pltpu.VMEM((BATCH, POS_PER_WORKER), jnp.int32),
            pltpu.VMEM((POS_PER_WORKER, HIDDEN), jnp.float32),
            pltpu.VMEM((NBUF, CHUNK, HIDDEN), jnp.float32),
            pltpu.SemaphoreType.DMA((NBUF,)),
            pltpu.SemaphoreType.DMA((NBUF,)),
        ],
    )(_emb_body)
    return k(ids_flat, token_table, pos_table)


def kernel(input_ids, token_table, pos_table):
    ids_flat = input_ids.reshape(-1).astype(jnp.int32)
    out = _emb(ids_flat, token_table, pos_table)
    return out.reshape(BATCH, SEQ, HIDDEN)
